# Initial kernel scaffold; baseline (speedup 1.0000x reference)
import jax
import jax.numpy as jnp
from jax.experimental import pallas as pl


def kernel(h, pos, edge_index, batch, params):
    raise NotImplementedError("write your pallas kernel here")


# speedup vs baseline: 2.2560x; 2.2560x over previous
import functools
from math import pi as PI

import jax
import jax.numpy as jnp
from jax import lax
from jax.experimental import pallas as pl
from jax.experimental.pallas import tpu as pltpu
from jax.experimental.pallas import tpu_sc as plsc

N = 10000
E = 320000
H = 128
NF = 128
NG = 50
NL = 6
CUTOFF = 10.0
NGRAPH = 64

NPAD = 10240
NW = 32
EPT = E // NW
CK = 80
NCHUNK = EPT // CK
NSL = NPAD // 16
BE = 512
BN = 512

_DELTA = CUTOFF / (NG - 1)
_COEFF = -0.5 / _DELTA**2
_F32 = jnp.float32


def _ssp(x):
    return jnp.logaddexp(x, 0.0) - jnp.log(2.0)


def _dot(a, b):
    return jnp.dot(a, b, preferred_element_type=_F32)


def _x1_body(h_ref, w_ref, o_ref):
    o_ref[...] = _dot(h_ref[...], w_ref[...])


def _x1_call(hP, w):
    return pl.pallas_call(
        _x1_body,
        grid=(NPAD // BN,),
        in_specs=[
            pl.BlockSpec((BN, H), lambda i: (i, 0)),
            pl.BlockSpec((H, NF), lambda i: (0, 0)),
        ],
        out_specs=pl.BlockSpec((BN, NF), lambda i: (i, 0)),
        out_shape=jax.ShapeDtypeStruct((NPAD, NF), _F32),
    )(hP, w)


def _edge_body(q_ref, off_ref, co_ref, w1_ref, b1_ref, w2_ref, b2_ref,
               cwa_ref, cb_ref, wf_ref, sea_ref):
    q = q_ref[...]
    ew = jnp.sqrt(q + 1e-12)
    d = ew - off_ref[...]
    ea = jnp.exp(co_ref[...] * (d * d))
    t = _ssp(_dot(ea, w1_ref[...]) + b1_ref[...])
    wf = _dot(t, w2_ref[...]) + b2_ref[...]
    cw = 0.5 * (jnp.cos(ew * PI / CUTOFF) + 1.0)
    wf_ref[...] = wf * cw
    sea_ref[...] = _dot(ea, cwa_ref[...]) + cb_ref[...]


def _edge_call(q, off, co, w1, b1, w2, b2, cwa, cb):
    return pl.pallas_call(
        _edge_body,
        grid=(E // BE,),
        in_specs=[
            pl.BlockSpec((BE, 1), lambda i: (i, 0)),
            pl.BlockSpec((1, NG), lambda i: (0, 0)),
            pl.BlockSpec((1, 1), lambda i: (0, 0)),
            pl.BlockSpec((NG, NF), lambda i: (0, 0)),
            pl.BlockSpec((1, NF), lambda i: (0, 0)),
            pl.BlockSpec((NF, NF), lambda i: (0, 0)),
            pl.BlockSpec((1, NF), lambda i: (0, 0)),
            pl.BlockSpec((NG, 1), lambda i: (0, 0)),
            pl.BlockSpec((1, 1), lambda i: (0, 0)),
        ],
        out_specs=[
            pl.BlockSpec((BE, NF), lambda i: (i, 0)),
            pl.BlockSpec((BE, 1), lambda i: (i, 0)),
        ],
        out_shape=[
            jax.ShapeDtypeStruct((E, NF), _F32),
            jax.ShapeDtypeStruct((E, 1), _F32),
        ],
    )(q, off, co, w1, b1, w2, b2, cwa, cb)


def _node_body(h_ref, a0_ref, a1_ref, l2w_ref, l2b_ref, lw_ref, lb_ref,
               c1n_ref, wb_ref, wc_ref, hn_ref, x1_ref, sb_ref, sc_ref):
    agg = a0_ref[...] + a1_ref[...]
    x2 = _dot(agg, l2w_ref[...]) + l2b_ref[...]
    dh = _dot(_ssp(x2), lw_ref[...]) + lb_ref[...]
    hn = h_ref[...] + dh
    hn_ref[...] = hn
    x1_ref[...] = _dot(hn, c1n_ref[...])
    sb_ref[...] = _dot(hn, wb_ref[...])
    sc_ref[...] = _dot(hn, wc_ref[...])


def _node_call(hP, a0, a1, l2w, l2b, lw, lb, c1n, wb, wc):
    return pl.pallas_call(
        _node_body,
        grid=(NPAD // BN,),
        in_specs=[
            pl.BlockSpec((BN, H), lambda i: (i, 0)),
            pl.BlockSpec((BN, NF), lambda i: (i, 0)),
            pl.BlockSpec((BN, NF), lambda i: (i, 0)),
            pl.BlockSpec((NF, H), lambda i: (0, 0)),
            pl.BlockSpec((1, H), lambda i: (0, 0)),
            pl.BlockSpec((H, H), lambda i: (0, 0)),
            pl.BlockSpec((1, H), lambda i: (0, 0)),
            pl.BlockSpec((H, NF), lambda i: (0, 0)),
            pl.BlockSpec((H, 1), lambda i: (0, 0)),
            pl.BlockSpec((H, 1), lambda i: (0, 0)),
        ],
        out_specs=[
            pl.BlockSpec((BN, H), lambda i: (i, 0)),
            pl.BlockSpec((BN, NF), lambda i: (i, 0)),
            pl.BlockSpec((BN, 1), lambda i: (i, 0)),
            pl.BlockSpec((BN, 1), lambda i: (i, 0)),
        ],
        out_shape=[
            jax.ShapeDtypeStruct((NPAD, H), _F32),
            jax.ShapeDtypeStruct((NPAD, NF), _F32),
            jax.ShapeDtypeStruct((NPAD, 1), _F32),
            jax.ShapeDtypeStruct((NPAD, 1), _F32),
        ],
    )(hP, a0, a1, l2w, l2b, lw, lb, c1n, wb, wc)


def _node5_body(h_ref, a0_ref, a1_ref, l2w_ref, l2b_ref, lw_ref, lb_ref,
                hn_ref):
    agg = a0_ref[...] + a1_ref[...]
    x2 = _dot(agg, l2w_ref[...]) + l2b_ref[...]
    dh = _dot(_ssp(x2), lw_ref[...]) + lb_ref[...]
    hn_ref[...] = h_ref[...] + dh


def _node5_call(hP, a0, a1, l2w, l2b, lw, lb):
    return pl.pallas_call(
        _node5_body,
        grid=(NPAD // BN,),
        in_specs=[
            pl.BlockSpec((BN, H), lambda i: (i, 0)),
            pl.BlockSpec((BN, NF), lambda i: (i, 0)),
            pl.BlockSpec((BN, NF), lambda i: (i, 0)),
            pl.BlockSpec((NF, H), lambda i: (0, 0)),
            pl.BlockSpec((1, H), lambda i: (0, 0)),
            pl.BlockSpec((H, H), lambda i: (0, 0)),
            pl.BlockSpec((1, H), lambda i: (0, 0)),
        ],
        out_specs=pl.BlockSpec((BN, H), lambda i: (i, 0)),
        out_shape=jax.ShapeDtypeStruct((NPAD, H), _F32),
    )(hP, a0, a1, l2w, l2b, lw, lb)


def _pos_body(px_ref, py_ref, pz_ref, n0x, n0y, n0z, n1x, n1y, n1z,
              c0_ref, c1_ref, pxo, pyo, pzo):
    c = jnp.maximum(c0_ref[...] + c1_ref[...], 1.0)
    pxo[...] = px_ref[...] + (n0x[...] + n1x[...]) / c
    pyo[...] = py_ref[...] + (n0y[...] + n1y[...]) / c
    pzo[...] = pz_ref[...] + (n0z[...] + n1z[...]) / c


def _pos_call(px, py, pz, num, cnt2):
    v = lambda a: a.reshape(NPAD // 128, 128)
    full = pl.BlockSpec((NPAD // 128, 128), lambda: (0, 0))
    outs = pl.pallas_call(
        _pos_body,
        in_specs=[full] * 11,
        out_specs=[full] * 3,
        out_shape=[jax.ShapeDtypeStruct((NPAD // 128, 128), _F32)] * 3,
    )(v(px), v(py), v(pz),
      v(num[0, 0]), v(num[0, 1]), v(num[0, 2]),
      v(num[1, 0]), v(num[1, 1]), v(num[1, 2]),
      v(cnt2[0]), v(cnt2[1]))
    return tuple(o.reshape(NPAD) for o in outs)


def _readout_body(h_ref, b_ref, w1_ref, b1_ref, w2_ref, b2_ref, o_ref):
    i = pl.program_id(0)
    o = _dot(_ssp(_dot(h_ref[...], w1_ref[...]) + b1_ref[...]), w2_ref[...]) \
        + b2_ref[...]
    gid = jax.lax.broadcasted_iota(jnp.int32, (BN, NGRAPH), 1)
    onehot = (b_ref[...] == gid).astype(_F32)
    row = i * BN + jax.lax.broadcasted_iota(jnp.int32, (BN, 1), 0)
    valid = (row < N).astype(_F32)
    contrib = jnp.sum(o * valid * onehot, axis=0, keepdims=True)

    @pl.when(i == 0)
    def _():
        o_ref[...] = jnp.zeros_like(o_ref)

    o_ref[...] += contrib


def _readout_call(hP, batchP, w1, b1, w2, b2):
    return pl.pallas_call(
        _readout_body,
        grid=(NPAD // BN,),
        in_specs=[
            pl.BlockSpec((BN, H), lambda i: (i, 0)),
            pl.BlockSpec((BN, 1), lambda i: (i, 0)),
            pl.BlockSpec((H, H // 2), lambda i: (0, 0)),
            pl.BlockSpec((1, H // 2), lambda i: (0, 0)),
            pl.BlockSpec((H // 2, 1), lambda i: (0, 0)),
            pl.BlockSpec((1, 1), lambda i: (0, 0)),
        ],
        out_specs=pl.BlockSpec((1, NGRAPH), lambda i: (0, 0)),
        out_shape=jax.ShapeDtypeStruct((1, NGRAPH), _F32),
    )(hP, batchP, w1, b1, w2, b2)


def _geom_body(px_h, py_h, pz_h, src_h, dst_h,
               cdx_h, cdy_h, cdz_h, q_h,
               pxv, pyv, pzv, siv, div, ox, oy, oz, oq):
    wid = lax.axis_index("c") * 16 + lax.axis_index("s")
    pltpu.sync_copy(px_h, pxv)
    pltpu.sync_copy(py_h, pyv)
    pltpu.sync_copy(pz_h, pzv)

    def chunk(c, carry):
        b = wid * EPT + c * CK
        pltpu.sync_copy(src_h.at[pl.ds(b, CK)], siv)
        pltpu.sync_copy(dst_h.at[pl.ds(b, CK)], div)
        for g in range(CK // 16):
            sl = pl.ds(g * 16, 16)
            s16 = siv[sl]
            d16 = div[sl]
            cx = plsc.load_gather(pxv, [s16]) - plsc.load_gather(pxv, [d16])
            cy = plsc.load_gather(pyv, [s16]) - plsc.load_gather(pyv, [d16])
            cz = plsc.load_gather(pzv, [s16]) - plsc.load_gather(pzv, [d16])
            ox[sl] = cx
            oy[sl] = cy
            oz[sl] = cz
            oq[sl] = cx * cx + cy * cy + cz * cz
        pltpu.sync_copy(ox, cdx_h.at[pl.ds(b, CK)])
        pltpu.sync_copy(oy, cdy_h.at[pl.ds(b, CK)])
        pltpu.sync_copy(oz, cdz_h.at[pl.ds(b, CK)])
        pltpu.sync_copy(oq, q_h.at[pl.ds(b, CK)])
        return carry

    lax.fori_loop(0, NCHUNK, chunk, 0)


def _zero_1d(ref, n):
    zz = jnp.zeros((16,), _F32)

    def z(k, carry):
        ref[pl.ds(k * 16, 16)] = zz
        return carry

    lax.fori_loop(0, n // 16, z, 0)


def _deg_body(src_h, out_h, siv, onesb, accv, shared):
    cid = lax.axis_index("c")
    sid = lax.axis_index("s")
    wid = cid * 16 + sid
    base = sid * NSL
    _zero_1d(accv, NSL)
    pltpu.sync_copy(accv, shared.at[pl.ds(base, NSL)])

    def fill(k, carry):
        onesb[pl.ds(k * 16, 16)] = jnp.ones((16,), _F32)
        return carry

    lax.fori_loop(0, CK // 16, fill, 0)
    plsc.subcore_barrier()

    def chunk(c, carry):
        b = wid * EPT + c * CK
        pltpu.sync_copy(src_h.at[pl.ds(b, CK)], siv)
        pltpu.sync_copy(onesb, shared.at[siv], add=True)
        return carry

    lax.fori_loop(0, NCHUNK, chunk, 0)
    plsc.subcore_barrier()
    pltpu.sync_copy(shared.at[pl.ds(base, NSL)], accv)
    pltpu.sync_copy(accv, out_h.at[pl.ds(cid * NPAD + base, NSL)])


def _agg_body(x1_h, wf_h, src_h, dst_h, out_h, siv, div, xb, wfb, mb, sem,
              shared):
    cid = lax.axis_index("c")
    sid = lax.axis_index("s")
    wid = cid * 16 + sid
    zz = jnp.zeros((16,), _F32)

    def zr(r, carry):
        for u in range(NF // 16):
            mb[r, pl.ds(u * 16, 16)] = zz
        return carry

    lax.fori_loop(0, CK, zr, 0)
    for k in range(NSL // CK):
        pltpu.sync_copy(mb, shared.at[pl.ds(sid * NSL + k * CK, CK), :])
    plsc.subcore_barrier()

    def chunk(c, carry):
        b = wid * EPT + c * CK
        pltpu.sync_copy(src_h.at[pl.ds(b, CK)], siv)
        pltpu.sync_copy(dst_h.at[pl.ds(b, CK)], div)
        pltpu.async_copy(x1_h.at[siv], xb, sem).wait()
        pltpu.sync_copy(wf_h.at[pl.ds(b, CK), :], wfb)

        def mul(r, carry2):
            for u in range(NF // 16):
                sl = pl.ds(u * 16, 16)
                mb[r, sl] = xb[r, sl] * wfb[r, sl]
            return carry2

        lax.fori_loop(0, CK, mul, 0)
        pltpu.sync_copy(mb, shared.at[div], add=True)
        return carry

    lax.fori_loop(0, NCHUNK, chunk, 0)
    plsc.subcore_barrier()
    for k in range(NSL // CK):
        rows = pl.ds(sid * NSL + k * CK, CK)
        pltpu.sync_copy(shared.at[rows, :], xb)
        rows_o = pl.ds(cid * NPAD + sid * NSL + k * CK, CK)
        pltpu.sync_copy(xb, out_h.at[rows_o, :])


def _coord_body(cdx_h, cdy_h, cdz_h, sea_h, sb_h, sc_h, src_h, dst_h, num_h,
                sbv, scv, siv, div, bx, by, bz, bs, accv,
                sharedx, sharedy, sharedz):
    cid = lax.axis_index("c")
    sid = lax.axis_index("s")
    wid = cid * 16 + sid
    base = sid * NSL
    pltpu.sync_copy(sb_h, sbv)
    pltpu.sync_copy(sc_h, scv)
    _zero_1d(accv, NSL)
    pltpu.sync_copy(accv, sharedx.at[pl.ds(base, NSL)])
    pltpu.sync_copy(accv, sharedy.at[pl.ds(base, NSL)])
    pltpu.sync_copy(accv, sharedz.at[pl.ds(base, NSL)])
    plsc.subcore_barrier()

    def chunk(c, carry):
        b = wid * EPT + c * CK
        pltpu.sync_copy(src_h.at[pl.ds(b, CK)], siv)
        pltpu.sync_copy(dst_h.at[pl.ds(b, CK)], div)
        pltpu.sync_copy(cdx_h.at[pl.ds(b, CK)], bx)
        pltpu.sync_copy(cdy_h.at[pl.ds(b, CK)], by)
        pltpu.sync_copy(cdz_h.at[pl.ds(b, CK)], bz)
        pltpu.sync_copy(sea_h.at[pl.ds(b, CK)], bs)
        for g in range(CK // 16):
            sl = pl.ds(g * 16, 16)
            s16 = siv[sl]
            d16 = div[sl]
            s = bs[sl] + plsc.load_gather(sbv, [s16]) \
                + plsc.load_gather(scv, [d16])
            bx[sl] = bx[sl] * s
            by[sl] = by[sl] * s
            bz[sl] = bz[sl] * s
        pltpu.sync_copy(bx, sharedx.at[siv], add=True)
        pltpu.sync_copy(by, sharedy.at[siv], add=True)
        pltpu.sync_copy(bz, sharedz.at[siv], add=True)
        return carry

    lax.fori_loop(0, NCHUNK, chunk, 0)
    plsc.subcore_barrier()
    for k, sh in enumerate((sharedx, sharedy, sharedz)):
        pltpu.sync_copy(sh.at[pl.ds(base, NSL)], accv)
        pltpu.sync_copy(accv, num_h.at[pl.ds(cid * 3 * NPAD + k * NPAD + base, NSL)])


@functools.lru_cache(maxsize=None)
def _sc_kernels():
    mesh = plsc.VectorSubcoreMesh(core_axis_name="c", subcore_axis_name="s",
                                  num_cores=2, num_subcores=16)
    geom = functools.partial(
        pl.kernel,
        out_type=[jax.ShapeDtypeStruct((E,), _F32)] * 4,
        mesh=mesh,
        compiler_params=pltpu.CompilerParams(needs_layout_passes=False),
        scratch_types=[
            pltpu.VMEM((NPAD,), _F32),
            pltpu.VMEM((NPAD,), _F32),
            pltpu.VMEM((NPAD,), _F32),
            pltpu.VMEM((CK,), jnp.int32),
            pltpu.VMEM((CK,), jnp.int32),
            pltpu.VMEM((CK,), _F32),
            pltpu.VMEM((CK,), _F32),
            pltpu.VMEM((CK,), _F32),
            pltpu.VMEM((CK,), _F32),
        ],
    )(_geom_body)
    deg = functools.partial(
        pl.kernel,
        out_type=jax.ShapeDtypeStruct((2 * NPAD,), _F32),
        mesh=mesh,
        compiler_params=pltpu.CompilerParams(needs_layout_passes=False),
        scratch_types=[
            pltpu.VMEM((CK,), jnp.int32),
            pltpu.VMEM((CK,), _F32),
            pltpu.VMEM((NSL,), _F32),
            pltpu.VMEM_SHARED((NPAD,), _F32),
        ],
    )(_deg_body)
    agg = functools.partial(
        pl.kernel,
        out_type=jax.ShapeDtypeStruct((2 * NPAD, NF), _F32),
        mesh=mesh,
        compiler_params=pltpu.CompilerParams(needs_layout_passes=False),
        scratch_types=[
            pltpu.VMEM((CK,), jnp.int32),
            pltpu.VMEM((CK,), jnp.int32),
            pltpu.VMEM((CK, NF), _F32),
            pltpu.VMEM((CK, NF), _F32),
            pltpu.VMEM((CK, NF), _F32),
            pltpu.SemaphoreType.DMA,
            pltpu.VMEM_SHARED((NPAD, NF), _F32),
        ],
    )(_agg_body)
    coord = functools.partial(
        pl.kernel,
        out_type=jax.ShapeDtypeStruct((2 * 3 * NPAD,), _F32),
        mesh=mesh,
        compiler_params=pltpu.CompilerParams(needs_layout_passes=False),
        scratch_types=[
            pltpu.VMEM((NPAD,), _F32),
            pltpu.VMEM((NPAD,), _F32),
            pltpu.VMEM((CK,), jnp.int32),
            pltpu.VMEM((CK,), jnp.int32),
            pltpu.VMEM((CK,), _F32),
            pltpu.VMEM((CK,), _F32),
            pltpu.VMEM((CK,), _F32),
            pltpu.VMEM((CK,), _F32),
            pltpu.VMEM((NSL,), _F32),
            pltpu.VMEM_SHARED((NPAD,), _F32),
            pltpu.VMEM_SHARED((NPAD,), _F32),
            pltpu.VMEM_SHARED((NPAD,), _F32),
        ],
    )(_coord_body)
    return geom, deg, agg, coord


def kernel(h, pos, edge_index, batch, params):
    src = edge_index[0].astype(jnp.int32)
    dst = edge_index[1].astype(jnp.int32)
    pad_n = NPAD - N
    px = jnp.pad(pos[:, 0], (0, pad_n))
    py = jnp.pad(pos[:, 1], (0, pad_n))
    pz = jnp.pad(pos[:, 2], (0, pad_n))
    hP = jnp.pad(h, ((0, pad_n), (0, 0)))
    batchP = jnp.pad(batch.astype(jnp.int32), (0, pad_n)).reshape(NPAD, 1)

    _geom_call, _deg_call, _agg_call, _coord_call = _sc_kernels()

    offset = jnp.linspace(0.0, CUTOFF, NG)
    coeff = -0.5 / (offset[1] - offset[0]) ** 2

    cnt2 = _deg_call(src).reshape(2, NPAD)
    x1 = _x1_call(hP, params["conv_lin1_0"])

    for i in range(NL):
        cdx, cdy, cdz, q = _geom_call(px, py, pz, src, dst)
        cw = params[f"coord_w_{i}"]
        wf, sea = _edge_call(
            q.reshape(E, 1), offset.reshape(1, NG), coeff.reshape(1, 1),
            params[f"mlp_w1_{i}"], params[f"mlp_b1_{i}"].reshape(1, NF),
            params[f"mlp_w2_{i}"], params[f"mlp_b2_{i}"].reshape(1, NF),
            cw[:NG], params[f"coord_b_{i}"].reshape(1, 1),
        )
        agg2 = _agg_call(x1, wf, src, dst).reshape(2, NPAD, NF)
        l2w = params[f"conv_lin2_w_{i}"]
        l2b = params[f"conv_lin2_b_{i}"].reshape(1, H)
        lw = params[f"lin_w_{i}"]
        lb = params[f"lin_b_{i}"].reshape(1, H)
        if i < NL - 1:
            hP, x1, sb, sc_ = _node_call(
                hP, agg2[0], agg2[1], l2w, l2b, lw, lb,
                params[f"conv_lin1_{i+1}"],
                cw[NG:NG + H], cw[NG + H:],
            )
            num = _coord_call(cdx, cdy, cdz, sea.reshape(E),
                              sb.reshape(NPAD), sc_.reshape(NPAD),
                              src, dst).reshape(2, 3, NPAD)
            px, py, pz = _pos_call(px, py, pz, num, cnt2)
        else:
            hP = _node5_call(hP, agg2[0], agg2[1], l2w, l2b, lw, lb)

    out = _readout_call(hP, batchP, params["out1_w"],
                        params["out1_b"].reshape(1, H // 2),
                        params["out2_w"], params["out2_b"].reshape(1, 1))
    return out.reshape(NGRAPH, 1)


# speedup vs baseline: 2.5843x; 1.1455x over previous
import functools
from math import pi as PI

import jax
import jax.numpy as jnp
from jax import lax
from jax.experimental import pallas as pl
from jax.experimental.pallas import tpu as pltpu
from jax.experimental.pallas import tpu_sc as plsc

N = 10000
E = 320000
H = 128
NF = 128
NG = 50
NL = 6
CUTOFF = 10.0
NGRAPH = 64

NPAD = 10240
NW = 32
EPT = E // NW
CK = 80
NCHUNK = EPT // CK
NSL = NPAD // 16
BE = 512
BN = 512

_DELTA = CUTOFF / (NG - 1)
_COEFF = -0.5 / _DELTA**2
_F32 = jnp.float32


def _ssp(x):
    return jnp.logaddexp(x, 0.0) - jnp.log(2.0)


def _dot(a, b):
    return jnp.dot(a, b, preferred_element_type=_F32)


def _x1_body(h_ref, w_ref, o_ref):
    o_ref[...] = _dot(h_ref[...], w_ref[...])


def _x1_call(hP, w):
    return pl.pallas_call(
        _x1_body,
        grid=(NPAD // BN,),
        in_specs=[
            pl.BlockSpec((BN, H), lambda i: (i, 0)),
            pl.BlockSpec((H, NF), lambda i: (0, 0)),
        ],
        out_specs=pl.BlockSpec((BN, NF), lambda i: (i, 0)),
        out_shape=jax.ShapeDtypeStruct((NPAD, NF), _F32),
    )(hP, w)


def _edge_body(q_ref, off_ref, co_ref, w1_ref, b1_ref, w2_ref, b2_ref,
               cwa_ref, cb_ref, wf_ref, sea_ref):
    q = q_ref[...]
    ew = jnp.sqrt(q + 1e-12)
    d = ew - off_ref[...]
    ea = jnp.exp(co_ref[...] * (d * d))
    t = _ssp(_dot(ea, w1_ref[...]) + b1_ref[...])
    wf = _dot(t, w2_ref[...]) + b2_ref[...]
    cw = 0.5 * (jnp.cos(ew * PI / CUTOFF) + 1.0)
    wf_ref[...] = wf * cw
    sea_ref[...] = _dot(ea, cwa_ref[...]) + cb_ref[...]


def _edge_call(q, off, co, w1, b1, w2, b2, cwa, cb):
    return pl.pallas_call(
        _edge_body,
        grid=(E // BE,),
        in_specs=[
            pl.BlockSpec((BE, 1), lambda i: (i, 0)),
            pl.BlockSpec((1, NG), lambda i: (0, 0)),
            pl.BlockSpec((1, 1), lambda i: (0, 0)),
            pl.BlockSpec((NG, NF), lambda i: (0, 0)),
            pl.BlockSpec((1, NF), lambda i: (0, 0)),
            pl.BlockSpec((NF, NF), lambda i: (0, 0)),
            pl.BlockSpec((1, NF), lambda i: (0, 0)),
            pl.BlockSpec((NG, 1), lambda i: (0, 0)),
            pl.BlockSpec((1, 1), lambda i: (0, 0)),
        ],
        out_specs=[
            pl.BlockSpec((BE, NF), lambda i: (i, 0)),
            pl.BlockSpec((BE, 1), lambda i: (i, 0)),
        ],
        out_shape=[
            jax.ShapeDtypeStruct((E, NF), _F32),
            jax.ShapeDtypeStruct((E, 1), _F32),
        ],
    )(q, off, co, w1, b1, w2, b2, cwa, cb)


def _node_body(h_ref, a0_ref, a1_ref, l2w_ref, l2b_ref, lw_ref, lb_ref,
               c1n_ref, wb_ref, wc_ref, hn_ref, x1_ref, sb_ref, sc_ref):
    agg = a0_ref[...] + a1_ref[...]
    x2 = _dot(agg, l2w_ref[...]) + l2b_ref[...]
    dh = _dot(_ssp(x2), lw_ref[...]) + lb_ref[...]
    hn = h_ref[...] + dh
    hn_ref[...] = hn
    x1_ref[...] = _dot(hn, c1n_ref[...])
    sb_ref[...] = _dot(hn, wb_ref[...])
    sc_ref[...] = _dot(hn, wc_ref[...])


def _node_call(hP, a0, a1, l2w, l2b, lw, lb, c1n, wb, wc):
    return pl.pallas_call(
        _node_body,
        grid=(NPAD // BN,),
        in_specs=[
            pl.BlockSpec((BN, H), lambda i: (i, 0)),
            pl.BlockSpec((BN, NF), lambda i: (i, 0)),
            pl.BlockSpec((BN, NF), lambda i: (i, 0)),
            pl.BlockSpec((NF, H), lambda i: (0, 0)),
            pl.BlockSpec((1, H), lambda i: (0, 0)),
            pl.BlockSpec((H, H), lambda i: (0, 0)),
            pl.BlockSpec((1, H), lambda i: (0, 0)),
            pl.BlockSpec((H, NF), lambda i: (0, 0)),
            pl.BlockSpec((H, 1), lambda i: (0, 0)),
            pl.BlockSpec((H, 1), lambda i: (0, 0)),
        ],
        out_specs=[
            pl.BlockSpec((BN, H), lambda i: (i, 0)),
            pl.BlockSpec((BN, NF), lambda i: (i, 0)),
            pl.BlockSpec((BN, 1), lambda i: (i, 0)),
            pl.BlockSpec((BN, 1), lambda i: (i, 0)),
        ],
        out_shape=[
            jax.ShapeDtypeStruct((NPAD, H), _F32),
            jax.ShapeDtypeStruct((NPAD, NF), _F32),
            jax.ShapeDtypeStruct((NPAD, 1), _F32),
            jax.ShapeDtypeStruct((NPAD, 1), _F32),
        ],
    )(hP, a0, a1, l2w, l2b, lw, lb, c1n, wb, wc)


def _node5_body(h_ref, a0_ref, a1_ref, l2w_ref, l2b_ref, lw_ref, lb_ref,
                hn_ref):
    agg = a0_ref[...] + a1_ref[...]
    x2 = _dot(agg, l2w_ref[...]) + l2b_ref[...]
    dh = _dot(_ssp(x2), lw_ref[...]) + lb_ref[...]
    hn_ref[...] = h_ref[...] + dh


def _node5_call(hP, a0, a1, l2w, l2b, lw, lb):
    return pl.pallas_call(
        _node5_body,
        grid=(NPAD // BN,),
        in_specs=[
            pl.BlockSpec((BN, H), lambda i: (i, 0)),
            pl.BlockSpec((BN, NF), lambda i: (i, 0)),
            pl.BlockSpec((BN, NF), lambda i: (i, 0)),
            pl.BlockSpec((NF, H), lambda i: (0, 0)),
            pl.BlockSpec((1, H), lambda i: (0, 0)),
            pl.BlockSpec((H, H), lambda i: (0, 0)),
            pl.BlockSpec((1, H), lambda i: (0, 0)),
        ],
        out_specs=pl.BlockSpec((BN, H), lambda i: (i, 0)),
        out_shape=jax.ShapeDtypeStruct((NPAD, H), _F32),
    )(hP, a0, a1, l2w, l2b, lw, lb)


def _pos_body(px_ref, py_ref, pz_ref, n0x, n0y, n0z, n1x, n1y, n1z,
              c0_ref, c1_ref, pxo, pyo, pzo):
    c = jnp.maximum(c0_ref[...] + c1_ref[...], 1.0)
    pxo[...] = px_ref[...] + (n0x[...] + n1x[...]) / c
    pyo[...] = py_ref[...] + (n0y[...] + n1y[...]) / c
    pzo[...] = pz_ref[...] + (n0z[...] + n1z[...]) / c


def _pos_call(px, py, pz, num, cnt2):
    v = lambda a: a.reshape(NPAD // 128, 128)
    full = pl.BlockSpec((NPAD // 128, 128), lambda: (0, 0))
    outs = pl.pallas_call(
        _pos_body,
        in_specs=[full] * 11,
        out_specs=[full] * 3,
        out_shape=[jax.ShapeDtypeStruct((NPAD // 128, 128), _F32)] * 3,
    )(v(px), v(py), v(pz),
      v(num[0, 0]), v(num[0, 1]), v(num[0, 2]),
      v(num[1, 0]), v(num[1, 1]), v(num[1, 2]),
      v(cnt2[0]), v(cnt2[1]))
    return tuple(o.reshape(NPAD) for o in outs)


def _readout_body(h_ref, b_ref, w1_ref, b1_ref, w2_ref, b2_ref, o_ref):
    i = pl.program_id(0)
    o = _dot(_ssp(_dot(h_ref[...], w1_ref[...]) + b1_ref[...]), w2_ref[...]) \
        + b2_ref[...]
    gid = jax.lax.broadcasted_iota(jnp.int32, (BN, NGRAPH), 1)
    onehot = (b_ref[...] == gid).astype(_F32)
    row = i * BN + jax.lax.broadcasted_iota(jnp.int32, (BN, 1), 0)
    valid = (row < N).astype(_F32)
    contrib = jnp.sum(o * valid * onehot, axis=0, keepdims=True)

    @pl.when(i == 0)
    def _():
        o_ref[...] = jnp.zeros_like(o_ref)

    o_ref[...] += contrib


def _readout_call(hP, batchP, w1, b1, w2, b2):
    return pl.pallas_call(
        _readout_body,
        grid=(NPAD // BN,),
        in_specs=[
            pl.BlockSpec((BN, H), lambda i: (i, 0)),
            pl.BlockSpec((BN, 1), lambda i: (i, 0)),
            pl.BlockSpec((H, H // 2), lambda i: (0, 0)),
            pl.BlockSpec((1, H // 2), lambda i: (0, 0)),
            pl.BlockSpec((H // 2, 1), lambda i: (0, 0)),
            pl.BlockSpec((1, 1), lambda i: (0, 0)),
        ],
        out_specs=pl.BlockSpec((1, NGRAPH), lambda i: (0, 0)),
        out_shape=jax.ShapeDtypeStruct((1, NGRAPH), _F32),
    )(hP, batchP, w1, b1, w2, b2)


def _geom_body(px_h, py_h, pz_h, src_h, dst_h,
               cdx_h, cdy_h, cdz_h, q_h,
               pxv, pyv, pzv, siv, div, ox, oy, oz, oq):
    wid = lax.axis_index("c") * 16 + lax.axis_index("s")
    pltpu.sync_copy(px_h, pxv)
    pltpu.sync_copy(py_h, pyv)
    pltpu.sync_copy(pz_h, pzv)

    def chunk(c, carry):
        b = wid * EPT + c * CK
        pltpu.sync_copy(src_h.at[pl.ds(b, CK)], siv)
        pltpu.sync_copy(dst_h.at[pl.ds(b, CK)], div)
        for g in range(CK // 16):
            sl = pl.ds(g * 16, 16)
            s16 = siv[sl]
            d16 = div[sl]
            cx = plsc.load_gather(pxv, [s16]) - plsc.load_gather(pxv, [d16])
            cy = plsc.load_gather(pyv, [s16]) - plsc.load_gather(pyv, [d16])
            cz = plsc.load_gather(pzv, [s16]) - plsc.load_gather(pzv, [d16])
            ox[sl] = cx
            oy[sl] = cy
            oz[sl] = cz
            oq[sl] = cx * cx + cy * cy + cz * cz
        pltpu.sync_copy(ox, cdx_h.at[pl.ds(b, CK)])
        pltpu.sync_copy(oy, cdy_h.at[pl.ds(b, CK)])
        pltpu.sync_copy(oz, cdz_h.at[pl.ds(b, CK)])
        pltpu.sync_copy(oq, q_h.at[pl.ds(b, CK)])
        return carry

    lax.fori_loop(0, NCHUNK, chunk, 0)


def _zero_1d(ref, n):
    zz = jnp.zeros((16,), _F32)

    def z(k, carry):
        ref[pl.ds(k * 16, 16)] = zz
        return carry

    lax.fori_loop(0, n // 16, z, 0)


def _deg_body(src_h, out_h, siv, onesb, accv, shared):
    cid = lax.axis_index("c")
    sid = lax.axis_index("s")
    wid = cid * 16 + sid
    base = sid * NSL
    _zero_1d(accv, NSL)
    pltpu.sync_copy(accv, shared.at[pl.ds(base, NSL)])

    def fill(k, carry):
        onesb[pl.ds(k * 16, 16)] = jnp.ones((16,), _F32)
        return carry

    lax.fori_loop(0, CK // 16, fill, 0)
    plsc.subcore_barrier()

    def chunk(c, carry):
        b = wid * EPT + c * CK
        pltpu.sync_copy(src_h.at[pl.ds(b, CK)], siv)
        pltpu.sync_copy(onesb, shared.at[siv], add=True)
        return carry

    lax.fori_loop(0, NCHUNK, chunk, 0)
    plsc.subcore_barrier()
    pltpu.sync_copy(shared.at[pl.ds(base, NSL)], accv)
    pltpu.sync_copy(accv, out_h.at[pl.ds(cid * NPAD + base, NSL)])


def _agg_body(x1_h, wf_h, src_h, dst_h, out_h, siv, div, siv2, div2,
              xb, wfb, xb2, wfb2, sem, sem2, sem3, sem4, shared):
    cid = lax.axis_index("c")
    sid = lax.axis_index("s")
    wid = cid * 16 + sid
    zz = jnp.zeros((16,), _F32)

    def zr(r, carry):
        for u in range(NF // 16):
            xb[r, pl.ds(u * 16, 16)] = zz
        return carry

    lax.fori_loop(0, CK, zr, 0)
    for k in range(NSL // CK):
        pltpu.sync_copy(xb, shared.at[pl.ds(sid * NSL + k * CK, CK), :])
    plsc.subcore_barrier()

    def issue(c, sib, dib, xbb, wfbb, gsem, lsem):
        b = wid * EPT + c * CK
        pltpu.sync_copy(src_h.at[pl.ds(b, CK)], sib)
        pltpu.sync_copy(dst_h.at[pl.ds(b, CK)], dib)
        pltpu.async_copy(x1_h.at[sib], xbb, gsem)
        pltpu.async_copy(wf_h.at[pl.ds(b, CK), :], wfbb, lsem)

    def process(c, sib, dib, xbb, wfbb, gsem, lsem):
        pltpu.make_async_copy(x1_h.at[sib], xbb, gsem).wait()
        b = wid * EPT + c * CK
        pltpu.make_async_copy(wf_h.at[pl.ds(b, CK), :], wfbb, lsem).wait()

        def mul(r, carry2):
            for u in range(NF // 16):
                sl = pl.ds(u * 16, 16)
                xbb[r, sl] = xbb[r, sl] * wfbb[r, sl]
            return carry2

        lax.fori_loop(0, CK, mul, 0)
        pltpu.sync_copy(xbb, shared.at[dib], add=True)

    issue(0, siv, div, xb, wfb, sem, sem2)

    def pair(c2, carry):
        c = 2 * c2
        issue(c + 1, siv2, div2, xb2, wfb2, sem3, sem4)
        process(c, siv, div, xb, wfb, sem, sem2)

        @pl.when(c + 2 < NCHUNK)
        def _():
            issue(c + 2, siv, div, xb, wfb, sem, sem2)

        process(c + 1, siv2, div2, xb2, wfb2, sem3, sem4)
        return carry

    lax.fori_loop(0, NCHUNK // 2, pair, 0)
    process(NCHUNK - 1, siv, div, xb, wfb, sem, sem2)
    plsc.subcore_barrier()
    for k in range(NSL // CK):
        rows = pl.ds(sid * NSL + k * CK, CK)
        pltpu.sync_copy(shared.at[rows, :], xb)
        rows_o = pl.ds(cid * NPAD + sid * NSL + k * CK, CK)
        pltpu.sync_copy(xb, out_h.at[rows_o, :])


def _coord_body(cdx_h, cdy_h, cdz_h, sea_h, sb_h, sc_h, src_h, dst_h, num_h,
                sbv, scv, siv, div, bx, by, bz, bs, accv,
                sharedx, sharedy, sharedz):
    cid = lax.axis_index("c")
    sid = lax.axis_index("s")
    wid = cid * 16 + sid
    base = sid * NSL
    pltpu.sync_copy(sb_h, sbv)
    pltpu.sync_copy(sc_h, scv)
    _zero_1d(accv, NSL)
    pltpu.sync_copy(accv, sharedx.at[pl.ds(base, NSL)])
    pltpu.sync_copy(accv, sharedy.at[pl.ds(base, NSL)])
    pltpu.sync_copy(accv, sharedz.at[pl.ds(base, NSL)])
    plsc.subcore_barrier()

    def chunk(c, carry):
        b = wid * EPT + c * CK
        pltpu.sync_copy(src_h.at[pl.ds(b, CK)], siv)
        pltpu.sync_copy(dst_h.at[pl.ds(b, CK)], div)
        pltpu.sync_copy(cdx_h.at[pl.ds(b, CK)], bx)
        pltpu.sync_copy(cdy_h.at[pl.ds(b, CK)], by)
        pltpu.sync_copy(cdz_h.at[pl.ds(b, CK)], bz)
        pltpu.sync_copy(sea_h.at[pl.ds(b, CK)], bs)
        for g in range(CK // 16):
            sl = pl.ds(g * 16, 16)
            s16 = siv[sl]
            d16 = div[sl]
            s = bs[sl] + plsc.load_gather(sbv, [s16]) \
                + plsc.load_gather(scv, [d16])
            bx[sl] = bx[sl] * s
            by[sl] = by[sl] * s
            bz[sl] = bz[sl] * s
        pltpu.sync_copy(bx, sharedx.at[siv], add=True)
        pltpu.sync_copy(by, sharedy.at[siv], add=True)
        pltpu.sync_copy(bz, sharedz.at[siv], add=True)
        return carry

    lax.fori_loop(0, NCHUNK, chunk, 0)
    plsc.subcore_barrier()
    for k, sh in enumerate((sharedx, sharedy, sharedz)):
        pltpu.sync_copy(sh.at[pl.ds(base, NSL)], accv)
        pltpu.sync_copy(accv, num_h.at[pl.ds(cid * 3 * NPAD + k * NPAD + base, NSL)])


@functools.lru_cache(maxsize=None)
def _sc_kernels():
    mesh = plsc.VectorSubcoreMesh(core_axis_name="c", subcore_axis_name="s",
                                  num_cores=2, num_subcores=16)
    geom = functools.partial(
        pl.kernel,
        out_type=[jax.ShapeDtypeStruct((E,), _F32)] * 4,
        mesh=mesh,
        compiler_params=pltpu.CompilerParams(needs_layout_passes=False),
        scratch_types=[
            pltpu.VMEM((NPAD,), _F32),
            pltpu.VMEM((NPAD,), _F32),
            pltpu.VMEM((NPAD,), _F32),
            pltpu.VMEM((CK,), jnp.int32),
            pltpu.VMEM((CK,), jnp.int32),
            pltpu.VMEM((CK,), _F32),
            pltpu.VMEM((CK,), _F32),
            pltpu.VMEM((CK,), _F32),
            pltpu.VMEM((CK,), _F32),
        ],
    )(_geom_body)
    deg = functools.partial(
        pl.kernel,
        out_type=jax.ShapeDtypeStruct((2 * NPAD,), _F32),
        mesh=mesh,
        compiler_params=pltpu.CompilerParams(needs_layout_passes=False),
        scratch_types=[
            pltpu.VMEM((CK,), jnp.int32),
            pltpu.VMEM((CK,), _F32),
            pltpu.VMEM((NSL,), _F32),
            pltpu.VMEM_SHARED((NPAD,), _F32),
        ],
    )(_deg_body)
    agg = functools.partial(
        pl.kernel,
        out_type=jax.ShapeDtypeStruct((2 * NPAD, NF), _F32),
        mesh=mesh,
        compiler_params=pltpu.CompilerParams(needs_layout_passes=False),
        scratch_types=[
            pltpu.VMEM((CK,), jnp.int32),
            pltpu.VMEM((CK,), jnp.int32),
            pltpu.VMEM((CK,), jnp.int32),
            pltpu.VMEM((CK,), jnp.int32),
            pltpu.VMEM((CK, NF), _F32),
            pltpu.VMEM((CK, NF), _F32),
            pltpu.VMEM((CK, NF), _F32),
            pltpu.VMEM((CK, NF), _F32),
            pltpu.SemaphoreType.DMA,
            pltpu.SemaphoreType.DMA,
            pltpu.SemaphoreType.DMA,
            pltpu.SemaphoreType.DMA,
            pltpu.VMEM_SHARED((NPAD, NF), _F32),
        ],
    )(_agg_body)
    coord = functools.partial(
        pl.kernel,
        out_type=jax.ShapeDtypeStruct((2 * 3 * NPAD,), _F32),
        mesh=mesh,
        compiler_params=pltpu.CompilerParams(needs_layout_passes=False),
        scratch_types=[
            pltpu.VMEM((NPAD,), _F32),
            pltpu.VMEM((NPAD,), _F32),
            pltpu.VMEM((CK,), jnp.int32),
            pltpu.VMEM((CK,), jnp.int32),
            pltpu.VMEM((CK,), _F32),
            pltpu.VMEM((CK,), _F32),
            pltpu.VMEM((CK,), _F32),
            pltpu.VMEM((CK,), _F32),
            pltpu.VMEM((NSL,), _F32),
            pltpu.VMEM_SHARED((NPAD,), _F32),
            pltpu.VMEM_SHARED((NPAD,), _F32),
            pltpu.VMEM_SHARED((NPAD,), _F32),
        ],
    )(_coord_body)
    return geom, deg, agg, coord


def kernel(h, pos, edge_index, batch, params):
    src = edge_index[0].astype(jnp.int32)
    dst = edge_index[1].astype(jnp.int32)
    pad_n = NPAD - N
    px = jnp.pad(pos[:, 0], (0, pad_n))
    py = jnp.pad(pos[:, 1], (0, pad_n))
    pz = jnp.pad(pos[:, 2], (0, pad_n))
    hP = jnp.pad(h, ((0, pad_n), (0, 0)))
    batchP = jnp.pad(batch.astype(jnp.int32), (0, pad_n)).reshape(NPAD, 1)

    _geom_call, _deg_call, _agg_call, _coord_call = _sc_kernels()

    offset = jnp.linspace(0.0, CUTOFF, NG)
    coeff = -0.5 / (offset[1] - offset[0]) ** 2

    cnt2 = _deg_call(src).reshape(2, NPAD)
    x1 = _x1_call(hP, params["conv_lin1_0"])

    for i in range(NL):
        cdx, cdy, cdz, q = _geom_call(px, py, pz, src, dst)
        cw = params[f"coord_w_{i}"]
        wf, sea = _edge_call(
            q.reshape(E, 1), offset.reshape(1, NG), coeff.reshape(1, 1),
            params[f"mlp_w1_{i}"], params[f"mlp_b1_{i}"].reshape(1, NF),
            params[f"mlp_w2_{i}"], params[f"mlp_b2_{i}"].reshape(1, NF),
            cw[:NG], params[f"coord_b_{i}"].reshape(1, 1),
        )
        agg2 = _agg_call(x1, wf, src, dst).reshape(2, NPAD, NF)
        l2w = params[f"conv_lin2_w_{i}"]
        l2b = params[f"conv_lin2_b_{i}"].reshape(1, H)
        lw = params[f"lin_w_{i}"]
        lb = params[f"lin_b_{i}"].reshape(1, H)
        if i < NL - 1:
            hP, x1, sb, sc_ = _node_call(
                hP, agg2[0], agg2[1], l2w, l2b, lw, lb,
                params[f"conv_lin1_{i+1}"],
                cw[NG:NG + H], cw[NG + H:],
            )
            num = _coord_call(cdx, cdy, cdz, sea.reshape(E),
                              sb.reshape(NPAD), sc_.reshape(NPAD),
                              src, dst).reshape(2, 3, NPAD)
            px, py, pz = _pos_call(px, py, pz, num, cnt2)
        else:
            hP = _node5_call(hP, agg2[0], agg2[1], l2w, l2b, lw, lb)

    out = _readout_call(hP, batchP, params["out1_w"],
                        params["out1_b"].reshape(1, H // 2),
                        params["out2_w"], params["out2_b"].reshape(1, 1))
    return out.reshape(NGRAPH, 1)
